# Initial kernel scaffold; baseline (speedup 1.0000x reference)
import jax
import jax.numpy as jnp
from jax.experimental import pallas as pl


def kernel(log_seqs, time1_seqs, time2_seqs, item_pop1, item_pop2):
    raise NotImplementedError("write your pallas kernel here")


# speedup vs baseline: 12.2188x; 12.2188x over previous
import jax
import jax.numpy as jnp
from jax import lax
from jax.experimental import pallas as pl
from jax.experimental.pallas import tpu as pltpu
from jax.experimental.pallas import tpu_sc as plsc

NC = 2
NS = 16
NW = NC * NS

IDX_ROW = 128
ROWS_PER_CHUNK = 2
CHUNK = ROWS_PER_CHUNK * IDX_ROW


def _gather_body(idx_hbm, tab_hbm, out_hbm, idx_v, out_v, sem):
    wid = lax.axis_index("s") * NC + lax.axis_index("c")
    total_rows = idx_hbm.shape[0]
    rows_per_w = total_rows // NW
    n_chunks = rows_per_w // ROWS_PER_CHUNK
    row_base0 = wid * rows_per_w

    def chunk_body(ci, carry):
        row_base = row_base0 + ci * ROWS_PER_CHUNK
        pltpu.sync_copy(idx_hbm.at[pl.ds(row_base, ROWS_PER_CHUNK)], idx_v)
        descs = []
        for j in range(ROWS_PER_CHUNK):
            descs.append(pltpu.async_copy(
                tab_hbm.at[idx_v.at[j]],
                out_v.at[pl.ds(j * IDX_ROW, IDX_ROW)], sem))
        for d in descs:
            d.wait()
        pltpu.sync_copy(out_v, out_hbm.at[pl.ds(row_base * IDX_ROW, CHUNK)])
        return carry

    lax.fori_loop(0, n_chunks, chunk_body, 0)


def kernel(log_seqs, time1_seqs, time2_seqs, item_pop1, item_pop2):
    batch, hist = log_seqs.shape
    d1 = item_pop1.shape[1]
    d2 = item_pop2.shape[1]
    d = d1 + d2
    n_idx = batch * hist
    idx2d = log_seqs.astype(jnp.int32).reshape(n_idx // IDX_ROW, IDX_ROW)
    tabcat = jnp.concatenate([item_pop1, item_pop2], axis=-1)

    mesh = plsc.VectorSubcoreMesh(core_axis_name="c", subcore_axis_name="s",
                                  num_cores=NC, num_subcores=NS)
    run = pl.kernel(
        _gather_body,
        out_type=jax.ShapeDtypeStruct((n_idx, d), jnp.float32),
        mesh=mesh,
        scratch_types=[
            pltpu.VMEM((ROWS_PER_CHUNK, IDX_ROW), jnp.int32),
            pltpu.VMEM((CHUNK, d), jnp.float32),
            pltpu.SemaphoreType.DMA,
        ],
    )
    out = run(idx2d, tabcat)
    return out.reshape(batch, hist, d)


# speedup vs baseline: 15.5473x; 1.2724x over previous
import jax
import jax.numpy as jnp
from jax import lax
from jax.experimental import pallas as pl
from jax.experimental.pallas import tpu as pltpu
from jax.experimental.pallas import tpu_sc as plsc

NC = 2
NS = 16
NW = NC * NS

IDX_ROW = 128
ROWS_PER_CHUNK = 2
CHUNK = ROWS_PER_CHUNK * IDX_ROW
NBUF = 2


def _gather_body(idx_hbm, tab_hbm, out_hbm,
                 idx_v0, idx_v1, out_v0, out_v1,
                 gsem0, gsem1, wsem0, wsem1):
    wid = lax.axis_index("s") * NC + lax.axis_index("c")
    total_rows = idx_hbm.shape[0]
    rows_per_w = total_rows // NW
    n_chunks = rows_per_w // ROWS_PER_CHUNK
    n_outer = n_chunks // NBUF
    row_base0 = wid * rows_per_w

    idx_bufs = (idx_v0, idx_v1)
    out_bufs = (out_v0, out_v1)
    gsems = (gsem0, gsem1)
    wsems = (wsem0, wsem1)

    def outer(k, carry):
        gdescs = [[] for _ in range(NBUF)]
        for b in range(NBUF):
            row_base = row_base0 + (k * NBUF + b) * ROWS_PER_CHUNK

            @pl.when(k > 0)
            def _drain(b=b):
                pltpu.make_async_copy(
                    out_hbm.at[pl.ds(0, CHUNK)], out_bufs[b], wsems[b]).wait()

            pltpu.sync_copy(idx_hbm.at[pl.ds(row_base, ROWS_PER_CHUNK)],
                            idx_bufs[b])
            for j in range(ROWS_PER_CHUNK):
                gdescs[b].append(pltpu.async_copy(
                    tab_hbm.at[idx_bufs[b].at[j]],
                    out_bufs[b].at[pl.ds(j * IDX_ROW, IDX_ROW)], gsems[b]))
        for b in range(NBUF):
            row_base = row_base0 + (k * NBUF + b) * ROWS_PER_CHUNK
            for d in gdescs[b]:
                d.wait()
            pltpu.async_copy(out_bufs[b],
                             out_hbm.at[pl.ds(row_base * IDX_ROW, CHUNK)],
                             wsems[b])
        return carry

    lax.fori_loop(0, n_outer, outer, 0)
    for b in range(NBUF):
        pltpu.make_async_copy(
            out_hbm.at[pl.ds(0, CHUNK)], out_bufs[b], wsems[b]).wait()


def kernel(log_seqs, time1_seqs, time2_seqs, item_pop1, item_pop2):
    batch, hist = log_seqs.shape
    d1 = item_pop1.shape[1]
    d2 = item_pop2.shape[1]
    d = d1 + d2
    n_idx = batch * hist
    idx2d = log_seqs.astype(jnp.int32).reshape(n_idx // IDX_ROW, IDX_ROW)
    tabcat = jnp.concatenate([item_pop1, item_pop2], axis=-1)

    mesh = plsc.VectorSubcoreMesh(core_axis_name="c", subcore_axis_name="s",
                                  num_cores=NC, num_subcores=NS)
    run = pl.kernel(
        _gather_body,
        out_type=jax.ShapeDtypeStruct((n_idx, d), jnp.float32),
        mesh=mesh,
        scratch_types=[
            pltpu.VMEM((ROWS_PER_CHUNK, IDX_ROW), jnp.int32),
            pltpu.VMEM((ROWS_PER_CHUNK, IDX_ROW), jnp.int32),
            pltpu.VMEM((CHUNK, d), jnp.float32),
            pltpu.VMEM((CHUNK, d), jnp.float32),
            pltpu.SemaphoreType.DMA,
            pltpu.SemaphoreType.DMA,
            pltpu.SemaphoreType.DMA,
            pltpu.SemaphoreType.DMA,
        ],
    )
    out = run(idx2d, tabcat)
    return out.reshape(batch, hist, d)
